# Initial kernel scaffold; baseline (speedup 1.0000x reference)
import functools

import jax
import jax.numpy as jnp
from jax import lax
from jax.experimental import pallas as pl
from jax.experimental.pallas import tpu as pltpu
from jax.experimental.pallas import tpu_sc as plsc

N = 10000
E = 320000
F_IN = 128
HID = 128
C = 64

NC = 2
NS = 16
NW = NC * NS
LANES = 128
K = -(-E // (NW * LANES))
E_PAD = NW * K * LANES
N_PAD = 10016
CHUNK = N_PAD // NS

_MESH = plsc.VectorSubcoreMesh(core_axis_name="c", subcore_axis_name="s")


@functools.partial(
    pl.kernel,
    out_type=jax.ShapeDtypeStruct((NC, N_PAD), jnp.float32),
    mesh=_MESH,
    scratch_types=[
        pltpu.VMEM((K, LANES), jnp.int32),
        pltpu.VMEM((LANES,), jnp.float32),
        pltpu.VMEM_SHARED((N_PAD,), jnp.float32),
    ],
)
def _sc_degree(dst_hbm, zero_hbm, out_hbm, dst_v, ones_v, acc_sh):
    c = lax.axis_index("c")
    s = lax.axis_index("s")
    w = s * NC + c
    for i in range(LANES // 16):
        ones_v[pl.ds(i * 16, 16)] = jnp.full((16,), 1.0, jnp.float32)
    pltpu.sync_copy(zero_hbm.at[pl.ds(s * CHUNK, CHUNK)],
                    acc_sh.at[pl.ds(s * CHUNK, CHUNK)])
    plsc.subcore_barrier()
    pltpu.sync_copy(dst_hbm.at[w], dst_v)

    def body(j, carry):
        pltpu.sync_copy(ones_v, acc_sh.at[dst_v.at[j]], add=True)
        return carry

    lax.fori_loop(0, K, body, 0)
    plsc.subcore_barrier()
    pltpu.sync_copy(acc_sh.at[pl.ds(s * CHUNK, CHUNK)],
                    out_hbm.at[c, pl.ds(s * CHUNK, CHUNK)])


@functools.partial(
    pl.kernel,
    out_type=jax.ShapeDtypeStruct((NC, N_PAD, HID), jnp.float32),
    mesh=_MESH,
    scratch_types=[
        pltpu.VMEM((K, LANES), jnp.int32),
        pltpu.VMEM((K, LANES), jnp.int32),
        pltpu.VMEM((LANES, HID), jnp.float32),
        pltpu.VMEM_SHARED((N_PAD, HID), jnp.float32),
        pltpu.SemaphoreType.DMA,
    ],
)
def _sc_scatter(tbl_hbm, src_hbm, dst_hbm, zero_hbm, out_hbm,
                src_v, dst_v, rows_v, acc_sh, sem):
    c = lax.axis_index("c")
    s = lax.axis_index("s")
    w = s * NC + c
    pltpu.sync_copy(zero_hbm.at[pl.ds(s * CHUNK, CHUNK)],
                    acc_sh.at[pl.ds(s * CHUNK, CHUNK)])
    plsc.subcore_barrier()
    pltpu.sync_copy(src_hbm.at[w], src_v)
    pltpu.sync_copy(dst_hbm.at[w], dst_v)

    def body(j, carry):
        pltpu.async_copy(tbl_hbm.at[src_v.at[j]], rows_v, sem).wait()
        pltpu.sync_copy(rows_v, acc_sh.at[dst_v.at[j]], add=True)
        return carry

    lax.fori_loop(0, K, body, 0)
    plsc.subcore_barrier()
    pltpu.sync_copy(acc_sh.at[pl.ds(s * CHUNK, CHUNK)],
                    out_hbm.at[c, pl.ds(s * CHUNK, CHUNK)])


def _tc_front_body(deg_ref, x_ref, wi_ref, w0_ref, dis_ref, hw_ref):
    p = deg_ref[...]
    dis = lax.rsqrt(1.0 + p[0] + p[1])
    dis_ref[...] = dis
    h0 = jnp.dot(x_ref[...], wi_ref[...], preferred_element_type=jnp.float32)
    hw = jnp.dot(h0, w0_ref[...], preferred_element_type=jnp.float32)
    hw_ref[...] = hw * dis[:N, :]


def _tc_mid_body(acc_ref, self_ref, dis_ref, b_ref, w_ref, out_ref):
    a = acc_ref[0, :N, :] + acc_ref[1, :N, :] + self_ref[...]
    dis = dis_ref[:N, :]
    h = jnp.maximum(a * dis + b_ref[...], 0.0)
    hw = jnp.dot(h, w_ref[...], preferred_element_type=jnp.float32)
    out_ref[...] = hw * dis


def _tc_out_body(acc_ref, self_ref, dis_ref, b_ref, w_ref, out_ref):
    a = acc_ref[0, :N, :] + acc_ref[1, :N, :] + self_ref[...]
    h = jnp.maximum(a * dis_ref[:N, :] + b_ref[...], 0.0)
    z = jnp.dot(h, w_ref[...], preferred_element_type=jnp.float32)
    m = jnp.max(z, axis=1, keepdims=True)
    lse = jnp.log(jnp.sum(jnp.exp(z - m), axis=1, keepdims=True))
    out_ref[...] = z - m - lse


def kernel(x, edge_index, W_in, W0, b0, W1, b1, W_out):
    src = edge_index[0]
    dst = edge_index[1]
    pad = E_PAD - E
    src_b = jnp.concatenate([src, jnp.zeros((pad,), jnp.int32)]).reshape(NW, K, LANES)
    dst_b = jnp.concatenate([dst, jnp.full((pad,), N, jnp.int32)]).reshape(NW, K, LANES)
    zeros1 = jnp.zeros((N_PAD,), jnp.float32)
    zeros2 = jnp.zeros((N_PAD, HID), jnp.float32)

    deg_p = _sc_degree(dst_b, zeros1).reshape(NC, N_PAD, 1)

    dis, hw0 = pl.pallas_call(
        _tc_front_body,
        out_shape=(jax.ShapeDtypeStruct((N_PAD, 1), jnp.float32),
                   jax.ShapeDtypeStruct((N, HID), jnp.float32)),
    )(deg_p, x, W_in, W0)

    acc1 = _sc_scatter(hw0, src_b, dst_b, zeros2)

    hw1 = pl.pallas_call(
        _tc_mid_body,
        out_shape=jax.ShapeDtypeStruct((N, HID), jnp.float32),
    )(acc1, hw0, dis, b0.reshape(1, HID), W1)

    acc2 = _sc_scatter(hw1, src_b, dst_b, zeros2)

    out = pl.pallas_call(
        _tc_out_body,
        out_shape=jax.ShapeDtypeStruct((N, C), jnp.float32),
    )(acc2, hw1, dis, b1.reshape(1, HID), W_out)

    return out


# speedup vs baseline: 12.3162x; 12.3162x over previous
import functools

import jax
import jax.numpy as jnp
from jax import lax
from jax.experimental import pallas as pl
from jax.experimental.pallas import tpu as pltpu
from jax.experimental.pallas import tpu_sc as plsc

N = 10000
E = 320000
F_IN = 128
HID = 128
C = 64

NC = 2
NS = 16
NW = NC * NS
LANES = 128
K = -(-E // (NW * LANES))
E_PAD = NW * K * LANES
N_PAD = 10112
CHUNK = N_PAD // NS

_MESH = plsc.VectorSubcoreMesh(core_axis_name="c", subcore_axis_name="s")


DEGW = HID


@functools.partial(
    pl.kernel,
    out_type=jax.ShapeDtypeStruct((NC, N_PAD, DEGW), jnp.float32),
    mesh=_MESH,
    scratch_types=[
        pltpu.VMEM((K, LANES), jnp.int32),
        pltpu.VMEM((LANES, DEGW), jnp.float32),
        pltpu.VMEM_SHARED((N_PAD, DEGW), jnp.float32),
    ],
)
def _sc_degree(dst_hbm, zero_hbm, ones_hbm, out_hbm, dst_v, ones_v, acc_sh):
    c = lax.axis_index("c")
    s = lax.axis_index("s")
    w = s * NC + c
    pltpu.sync_copy(ones_hbm, ones_v)
    pltpu.sync_copy(zero_hbm.at[pl.ds(s * CHUNK, CHUNK)],
                    acc_sh.at[pl.ds(s * CHUNK, CHUNK)])
    plsc.subcore_barrier()
    pltpu.sync_copy(dst_hbm.at[w], dst_v)

    def body(j, carry):
        pltpu.sync_copy(ones_v, acc_sh.at[dst_v.at[j]], add=True)
        return carry

    lax.fori_loop(0, K, body, 0)
    plsc.subcore_barrier()
    pltpu.sync_copy(acc_sh.at[pl.ds(s * CHUNK, CHUNK)],
                    out_hbm.at[c, pl.ds(s * CHUNK, CHUNK)])


@functools.partial(
    pl.kernel,
    out_type=jax.ShapeDtypeStruct((NC, N_PAD, HID), jnp.float32),
    mesh=_MESH,
    scratch_types=[
        pltpu.VMEM((K, LANES), jnp.int32),
        pltpu.VMEM((K, LANES), jnp.int32),
        pltpu.VMEM((LANES, HID), jnp.float32),
        pltpu.VMEM_SHARED((N_PAD, HID), jnp.float32),
        pltpu.SemaphoreType.DMA,
    ],
)
def _sc_scatter(tbl_hbm, src_hbm, dst_hbm, zero_hbm, out_hbm,
                src_v, dst_v, rows_v, acc_sh, sem):
    c = lax.axis_index("c")
    s = lax.axis_index("s")
    w = s * NC + c
    pltpu.sync_copy(zero_hbm.at[pl.ds(s * CHUNK, CHUNK)],
                    acc_sh.at[pl.ds(s * CHUNK, CHUNK)])
    plsc.subcore_barrier()
    pltpu.sync_copy(src_hbm.at[w], src_v)
    pltpu.sync_copy(dst_hbm.at[w], dst_v)

    def body(j, carry):
        pltpu.async_copy(tbl_hbm.at[src_v.at[j]], rows_v, sem).wait()
        pltpu.sync_copy(rows_v, acc_sh.at[dst_v.at[j]], add=True)
        return carry

    lax.fori_loop(0, K, body, 0)
    plsc.subcore_barrier()
    pltpu.sync_copy(acc_sh.at[pl.ds(s * CHUNK, CHUNK)],
                    out_hbm.at[c, pl.ds(s * CHUNK, CHUNK)])


def _tc_front_body(deg_ref, x_ref, wi_ref, w0_ref, dis_ref, hw_ref):
    p = deg_ref[...]
    deg_col = p[0, :, :1] + p[1, :, :1]
    dis = lax.rsqrt(1.0 + deg_col)
    dis_ref[...] = dis
    h0 = jnp.dot(x_ref[...], wi_ref[...], preferred_element_type=jnp.float32)
    hw = jnp.dot(h0, w0_ref[...], preferred_element_type=jnp.float32)
    hw_ref[...] = hw * dis[:N, :]


def _tc_mid_body(acc_ref, self_ref, dis_ref, b_ref, w_ref, out_ref):
    a = acc_ref[0, :N, :] + acc_ref[1, :N, :] + self_ref[...]
    dis = dis_ref[:N, :]
    h = jnp.maximum(a * dis + b_ref[...], 0.0)
    hw = jnp.dot(h, w_ref[...], preferred_element_type=jnp.float32)
    out_ref[...] = hw * dis


def _tc_out_body(acc_ref, self_ref, dis_ref, b_ref, w_ref, out_ref):
    a = acc_ref[0, :N, :] + acc_ref[1, :N, :] + self_ref[...]
    h = jnp.maximum(a * dis_ref[:N, :] + b_ref[...], 0.0)
    z = jnp.dot(h, w_ref[...], preferred_element_type=jnp.float32)
    m = jnp.max(z, axis=1, keepdims=True)
    lse = jnp.log(jnp.sum(jnp.exp(z - m), axis=1, keepdims=True))
    out_ref[...] = z - m - lse


def kernel(x, edge_index, W_in, W0, b0, W1, b1, W_out):
    src = edge_index[0]
    dst = edge_index[1]
    pad = E_PAD - E
    src_b = jnp.concatenate([src, jnp.zeros((pad,), jnp.int32)]).reshape(NW, K, LANES)
    dst_b = jnp.concatenate([dst, jnp.full((pad,), N, jnp.int32)]).reshape(NW, K, LANES)
    zeros2 = jnp.zeros((N_PAD, HID), jnp.float32)
    zeros1 = zeros2
    ones1 = jnp.ones((LANES, DEGW), jnp.float32)

    deg_p = _sc_degree(dst_b, zeros1, ones1)

    dis, hw0 = pl.pallas_call(
        _tc_front_body,
        out_shape=(jax.ShapeDtypeStruct((N_PAD, 1), jnp.float32),
                   jax.ShapeDtypeStruct((N, HID), jnp.float32)),
    )(deg_p, x, W_in, W0)

    acc1 = _sc_scatter(hw0, src_b, dst_b, zeros2)

    hw1 = pl.pallas_call(
        _tc_mid_body,
        out_shape=jax.ShapeDtypeStruct((N, HID), jnp.float32),
    )(acc1, hw0, dis, b0.reshape(1, HID), W1)

    acc2 = _sc_scatter(hw1, src_b, dst_b, zeros2)

    out = pl.pallas_call(
        _tc_out_body,
        out_shape=jax.ShapeDtypeStruct((N, C), jnp.float32),
    )(acc2, hw1, dis, b1.reshape(1, HID), W_out)

    return out


# speedup vs baseline: 25.5035x; 2.0707x over previous
import functools

import jax
import jax.numpy as jnp
from jax import lax
from jax.experimental import pallas as pl
from jax.experimental.pallas import tpu as pltpu
from jax.experimental.pallas import tpu_sc as plsc

N = 10000
E = 320000
F_IN = 128
HID = 128
C = 64

NC = 2
NS = 16
NW = NC * NS
LANES = 128
K = 80
KH = K // 2
E_PAD = NW * K * LANES
N_PAD = 10112
CHUNK = N_PAD // NS

_MESH = plsc.VectorSubcoreMesh(core_axis_name="c", subcore_axis_name="s")


DEGW = HID


@functools.partial(
    pl.kernel,
    out_type=jax.ShapeDtypeStruct((NC, N_PAD, DEGW), jnp.float32),
    mesh=_MESH,
    scratch_types=[
        pltpu.VMEM((K, LANES), jnp.int32),
        pltpu.VMEM((LANES, DEGW), jnp.float32),
        pltpu.VMEM_SHARED((N_PAD, DEGW), jnp.float32),
        pltpu.SemaphoreType.DMA,
    ],
)
def _sc_degree(dst_hbm, zero_hbm, ones_hbm, out_hbm, dst_v, ones_v, acc_sh,
               sem):
    c = lax.axis_index("c")
    s = lax.axis_index("s")
    w = s * NC + c
    pltpu.sync_copy(ones_hbm, ones_v)
    pltpu.sync_copy(zero_hbm.at[pl.ds(s * CHUNK, CHUNK)],
                    acc_sh.at[pl.ds(s * CHUNK, CHUNK)])
    plsc.subcore_barrier()
    pltpu.sync_copy(dst_hbm.at[w], dst_v)

    def body(g, carry):
        descs = [
            pltpu.async_copy(ones_v, acc_sh.at[dst_v.at[g * 8 + b]], sem,
                             add=True)
            for b in range(8)
        ]
        for d in descs:
            d.wait()
        return carry

    lax.fori_loop(0, K // 8, body, 0)
    plsc.subcore_barrier()
    pltpu.sync_copy(acc_sh.at[pl.ds(s * CHUNK, CHUNK)],
                    out_hbm.at[c, pl.ds(s * CHUNK, CHUNK)])


@functools.partial(
    pl.kernel,
    out_type=jax.ShapeDtypeStruct((NC, N_PAD, HID), jnp.float32),
    mesh=_MESH,
    scratch_types=[
        pltpu.VMEM((K // 2, LANES), jnp.int32),
        pltpu.VMEM((K // 2, LANES), jnp.int32),
        pltpu.VMEM((LANES, HID), jnp.float32),
        pltpu.VMEM((LANES, HID), jnp.float32),
        pltpu.VMEM_SHARED((N_PAD, HID), jnp.float32),
        pltpu.SemaphoreType.DMA,
        pltpu.SemaphoreType.DMA,
        pltpu.SemaphoreType.DMA,
        pltpu.SemaphoreType.DMA,
    ],
)
def _sc_scatter(tbl_hbm, src_hbm, dst_hbm, zero_hbm, out_hbm,
                src_v, dst_v, rows_a, rows_b, acc_sh, sga, sgb, ssa, ssb):
    c = lax.axis_index("c")
    s = lax.axis_index("s")
    w = s * NC + c
    pltpu.sync_copy(zero_hbm.at[pl.ds(s * CHUNK, CHUNK)],
                    acc_sh.at[pl.ds(s * CHUNK, CHUNK)])
    plsc.subcore_barrier()

    GRP = 8
    KP = K // 2
    bufs = (rows_a, rows_b)
    gsem = (sga, sgb)
    ssem = (ssa, ssb)

    def body(grp, carry):
        j0 = grp * GRP
        g = [None] * GRP
        sd = [None] * GRP
        g[0] = pltpu.async_copy(tbl_hbm.at[src_v.at[j0]], bufs[0], gsem[0])
        g[1] = pltpu.async_copy(tbl_hbm.at[src_v.at[j0 + 1]], bufs[1], gsem[1])
        for b in range(GRP):
            p = b % 2
            g[b].wait()
            sd[b] = pltpu.async_copy(bufs[p], acc_sh.at[dst_v.at[j0 + b]],
                                     ssem[p], add=True)
            if b + 2 < GRP:
                sd[b].wait()
                g[b + 2] = pltpu.async_copy(tbl_hbm.at[src_v.at[j0 + b + 2]],
                                            bufs[p], gsem[p])
        sd[GRP - 2].wait()
        sd[GRP - 1].wait()
        return carry

    for ph in range(2):
        pltpu.sync_copy(src_hbm.at[w, pl.ds(ph * KP, KP)], src_v)
        pltpu.sync_copy(dst_hbm.at[w, pl.ds(ph * KP, KP)], dst_v)
        lax.fori_loop(0, KP // GRP, body, 0)
    plsc.subcore_barrier()
    pltpu.sync_copy(acc_sh.at[pl.ds(s * CHUNK, CHUNK)],
                    out_hbm.at[c, pl.ds(s * CHUNK, CHUNK)])


def _tc_front_body(deg_ref, x_ref, wi_ref, w0_ref, dis_ref, hw_ref):
    p = deg_ref[...]
    deg_col = p[0, :, :1] + p[1, :, :1]
    dis = lax.rsqrt(1.0 + deg_col)
    dis_ref[...] = dis
    h0 = jnp.dot(x_ref[...], wi_ref[...], preferred_element_type=jnp.float32)
    hw = jnp.dot(h0, w0_ref[...], preferred_element_type=jnp.float32)
    hw_ref[...] = hw * dis[:N, :]


def _tc_mid_body(acc_ref, self_ref, dis_ref, b_ref, w_ref, out_ref):
    a = acc_ref[0, :N, :] + acc_ref[1, :N, :] + self_ref[...]
    dis = dis_ref[:N, :]
    h = jnp.maximum(a * dis + b_ref[...], 0.0)
    hw = jnp.dot(h, w_ref[...], preferred_element_type=jnp.float32)
    out_ref[...] = hw * dis


def _tc_out_body(acc_ref, self_ref, dis_ref, b_ref, w_ref, out_ref):
    a = acc_ref[0, :N, :] + acc_ref[1, :N, :] + self_ref[...]
    h = jnp.maximum(a * dis_ref[:N, :] + b_ref[...], 0.0)
    z = jnp.dot(h, w_ref[...], preferred_element_type=jnp.float32)
    m = jnp.max(z, axis=1, keepdims=True)
    lse = jnp.log(jnp.sum(jnp.exp(z - m), axis=1, keepdims=True))
    out_ref[...] = z - m - lse


def kernel(x, edge_index, W_in, W0, b0, W1, b1, W_out):
    src = edge_index[0]
    dst = edge_index[1]
    pad = E_PAD - E
    pad_ar = jnp.arange(pad, dtype=jnp.int32)
    pad_src = (pad_ar * 127) % N
    pad_dst = N + pad_ar % (N_PAD - N)
    src_b = jnp.concatenate([src, pad_src]).reshape(NW, K, LANES)
    dst_b = jnp.concatenate([dst, pad_dst]).reshape(NW, K, LANES)
    zeros2 = jnp.zeros((N_PAD, HID), jnp.float32)
    zeros1 = zeros2
    ones1 = jnp.ones((LANES, DEGW), jnp.float32)

    deg_p = _sc_degree(dst_b, zeros1, ones1)

    dis, hw0 = pl.pallas_call(
        _tc_front_body,
        out_shape=(jax.ShapeDtypeStruct((N_PAD, 1), jnp.float32),
                   jax.ShapeDtypeStruct((N, HID), jnp.float32)),
    )(deg_p, x, W_in, W0)

    acc1 = _sc_scatter(hw0, src_b, dst_b, zeros2)

    hw1 = pl.pallas_call(
        _tc_mid_body,
        out_shape=jax.ShapeDtypeStruct((N, HID), jnp.float32),
    )(acc1, hw0, dis, b0.reshape(1, HID), W1)

    acc2 = _sc_scatter(hw1, src_b, dst_b, zeros2)

    out = pl.pallas_call(
        _tc_out_body,
        out_shape=jax.ShapeDtypeStruct((N, C), jnp.float32),
    )(acc2, hw1, dis, b1.reshape(1, HID), W_out)

    return out


# speedup vs baseline: 25.6840x; 1.0071x over previous
import functools

import jax
import jax.numpy as jnp
from jax import lax
from jax.experimental import pallas as pl
from jax.experimental.pallas import tpu as pltpu
from jax.experimental.pallas import tpu_sc as plsc

N = 10000
E = 320000
F_IN = 128
HID = 128
C = 64

NC = 2
NS = 16
NW = NC * NS
LANES = 128
K = 80
KH = K // 2
E_PAD = NW * K * LANES
N_PAD = 10112
CHUNK = N_PAD // NS

_MESH = plsc.VectorSubcoreMesh(core_axis_name="c", subcore_axis_name="s")


DEGW = HID


@functools.partial(
    pl.kernel,
    out_type=jax.ShapeDtypeStruct((NC, N_PAD, DEGW), jnp.float32),
    mesh=_MESH,
    scratch_types=[
        pltpu.VMEM((K, LANES), jnp.int32),
        pltpu.VMEM((LANES, DEGW), jnp.float32),
        pltpu.VMEM_SHARED((N_PAD, DEGW), jnp.float32),
        pltpu.SemaphoreType.DMA,
    ],
)
def _sc_degree(dst_hbm, zero_hbm, ones_hbm, out_hbm, dst_v, ones_v, acc_sh,
               sem):
    c = lax.axis_index("c")
    s = lax.axis_index("s")
    w = s * NC + c
    pltpu.sync_copy(ones_hbm, ones_v)
    pltpu.sync_copy(zero_hbm.at[pl.ds(s * CHUNK, CHUNK)],
                    acc_sh.at[pl.ds(s * CHUNK, CHUNK)])
    plsc.subcore_barrier()
    pltpu.sync_copy(dst_hbm.at[w], dst_v)

    def body(g, carry):
        descs = [
            pltpu.async_copy(ones_v, acc_sh.at[dst_v.at[g * 8 + b]], sem,
                             add=True)
            for b in range(8)
        ]
        for d in descs:
            d.wait()
        return carry

    lax.fori_loop(0, K // 8, body, 0)
    plsc.subcore_barrier()
    pltpu.sync_copy(acc_sh.at[pl.ds(s * CHUNK, CHUNK)],
                    out_hbm.at[c, pl.ds(s * CHUNK, CHUNK)])


@functools.partial(
    pl.kernel,
    out_type=jax.ShapeDtypeStruct((NC, N_PAD, HID), jnp.float32),
    mesh=_MESH,
    scratch_types=[
        pltpu.VMEM((K // 2, LANES), jnp.int32),
        pltpu.VMEM((K // 2, LANES), jnp.int32),
        pltpu.VMEM((LANES, HID), jnp.float32),
        pltpu.VMEM((LANES, HID), jnp.float32),
        pltpu.VMEM_SHARED((N_PAD, HID), jnp.float32),
        pltpu.SemaphoreType.DMA,
        pltpu.SemaphoreType.DMA,
        pltpu.SemaphoreType.DMA,
        pltpu.SemaphoreType.DMA,
    ],
)
def _sc_scatter(tbl_hbm, src_hbm, dst_hbm, zero_hbm, out_hbm,
                src_v, dst_v, rows_a, rows_b, acc_sh, sga, sgb, ssa, ssb):
    c = lax.axis_index("c")
    s = lax.axis_index("s")
    w = s * NC + c
    pltpu.sync_copy(zero_hbm.at[pl.ds(s * CHUNK, CHUNK)],
                    acc_sh.at[pl.ds(s * CHUNK, CHUNK)])
    plsc.subcore_barrier()

    GRP = 10
    KP = K // 2
    bufs = (rows_a, rows_b)
    gsem = (sga, sgb)
    ssem = (ssa, ssb)

    def body(grp, carry):
        j0 = grp * GRP
        g = [None] * GRP
        sd = [None] * GRP
        g[0] = pltpu.async_copy(tbl_hbm.at[src_v.at[j0]], bufs[0], gsem[0])
        g[1] = pltpu.async_copy(tbl_hbm.at[src_v.at[j0 + 1]], bufs[1], gsem[1])
        for b in range(GRP):
            p = b % 2
            g[b].wait()
            sd[b] = pltpu.async_copy(bufs[p], acc_sh.at[dst_v.at[j0 + b]],
                                     ssem[p], add=True)
            if b + 2 < GRP:
                sd[b].wait()
                g[b + 2] = pltpu.async_copy(tbl_hbm.at[src_v.at[j0 + b + 2]],
                                            bufs[p], gsem[p])
        sd[GRP - 2].wait()
        sd[GRP - 1].wait()
        return carry

    for ph in range(2):
        pltpu.sync_copy(src_hbm.at[w, pl.ds(ph * KP, KP)], src_v)
        pltpu.sync_copy(dst_hbm.at[w, pl.ds(ph * KP, KP)], dst_v)
        lax.fori_loop(0, KP // GRP, body, 0)
    plsc.subcore_barrier()
    pltpu.sync_copy(acc_sh.at[pl.ds(s * CHUNK, CHUNK)],
                    out_hbm.at[c, pl.ds(s * CHUNK, CHUNK)])


def _tc_mm2_body(x_ref, wi_ref, w0_ref, hw_ref):
    h0 = jnp.dot(x_ref[...], wi_ref[...], preferred_element_type=jnp.float32)
    hw_ref[...] = jnp.dot(h0, w0_ref[...], preferred_element_type=jnp.float32)


def _tc_scale_body(deg_ref, hw_ref, dis_ref, hws_ref):
    p = deg_ref[...]
    deg_col = p[0, :, :1] + p[1, :, :1]
    dis = lax.rsqrt(1.0 + deg_col)
    dis_ref[...] = dis
    hws_ref[...] = hw_ref[...] * dis[:N, :]


def _tc_mid_body(acc_ref, self_ref, dis_ref, b_ref, w_ref, out_ref):
    a = acc_ref[0, :N, :] + acc_ref[1, :N, :] + self_ref[...]
    dis = dis_ref[:N, :]
    h = jnp.maximum(a * dis + b_ref[...], 0.0)
    hw = jnp.dot(h, w_ref[...], preferred_element_type=jnp.float32)
    out_ref[...] = hw * dis


def _tc_out_body(acc_ref, self_ref, dis_ref, b_ref, w_ref, out_ref):
    a = acc_ref[0, :N, :] + acc_ref[1, :N, :] + self_ref[...]
    h = jnp.maximum(a * dis_ref[:N, :] + b_ref[...], 0.0)
    z = jnp.dot(h, w_ref[...], preferred_element_type=jnp.float32)
    m = jnp.max(z, axis=1, keepdims=True)
    lse = jnp.log(jnp.sum(jnp.exp(z - m), axis=1, keepdims=True))
    out_ref[...] = z - m - lse


def kernel(x, edge_index, W_in, W0, b0, W1, b1, W_out):
    src = edge_index[0]
    dst = edge_index[1]
    pad = E_PAD - E
    pad_ar = jnp.arange(pad, dtype=jnp.int32)
    pad_src = (pad_ar * 127) % N
    pad_dst = N + pad_ar % (N_PAD - N)
    src_b = jnp.concatenate([src, pad_src]).reshape(NW, K, LANES)
    dst_b = jnp.concatenate([dst, pad_dst]).reshape(NW, K, LANES)
    zeros2 = jnp.zeros((N_PAD, HID), jnp.float32)
    zeros1 = zeros2
    ones1 = jnp.ones((LANES, DEGW), jnp.float32)

    hw0_raw = pl.pallas_call(
        _tc_mm2_body,
        out_shape=jax.ShapeDtypeStruct((N, HID), jnp.float32),
    )(x, W_in, W0)

    deg_p = _sc_degree(dst_b, zeros1, ones1)

    dis, hw0 = pl.pallas_call(
        _tc_scale_body,
        out_shape=(jax.ShapeDtypeStruct((N_PAD, 1), jnp.float32),
                   jax.ShapeDtypeStruct((N, HID), jnp.float32)),
    )(deg_p, hw0_raw)

    acc1 = _sc_scatter(hw0, src_b, dst_b, zeros2)

    hw1 = pl.pallas_call(
        _tc_mid_body,
        out_shape=jax.ShapeDtypeStruct((N, HID), jnp.float32),
    )(acc1, hw0, dis, b0.reshape(1, HID), W1)

    acc2 = _sc_scatter(hw1, src_b, dst_b, zeros2)

    out = pl.pallas_call(
        _tc_out_body,
        out_shape=jax.ShapeDtypeStruct((N, C), jnp.float32),
    )(acc2, hw1, dis, b1.reshape(1, HID), W_out)

    return out


# speedup vs baseline: 30.1359x; 1.1733x over previous
import functools

import jax
import jax.numpy as jnp
from jax import lax
from jax.experimental import pallas as pl
from jax.experimental.pallas import tpu as pltpu
from jax.experimental.pallas import tpu_sc as plsc

N = 10000
E = 320000
F_IN = 128
HID = 128
C = 64

NC = 2
NS = 16
NW = NC * NS
LANES = 128
K = 80
KH = K // 2
E_PAD = NW * K * LANES
N_PAD = 10112
CHUNK = N_PAD // NS

_MESH = plsc.VectorSubcoreMesh(core_axis_name="c", subcore_axis_name="s")


@functools.partial(
    pl.kernel,
    out_type=jax.ShapeDtypeStruct((NC * N_PAD,), jnp.float32),
    mesh=_MESH,
    scratch_types=[
        pltpu.VMEM((K, LANES), jnp.int32),
        pltpu.VMEM((LANES,), jnp.float32),
        pltpu.VMEM((CHUNK + 8,), jnp.float32),
        pltpu.VMEM_SHARED((N_PAD,), jnp.float32),
        pltpu.SemaphoreType.DMA,
    ],
)
def _sc_degree(dst_hbm, out_hbm, dst_v, ones_v, stage_v, acc_sh, sem):
    c = lax.axis_index("c")
    s = lax.axis_index("s")
    w = s * NC + c
    for i in range(LANES // 16):
        ones_v[pl.ds(i * 16, 16)] = jnp.full((16,), 1.0, jnp.float32)
    for i in range((CHUNK + 8) // 16):
        stage_v[pl.ds(i * 16, 16)] = jnp.zeros((16,), jnp.float32)
    pltpu.sync_copy(stage_v.at[pl.ds(0, CHUNK)],
                    acc_sh.at[pl.ds(s * CHUNK, CHUNK)])
    plsc.subcore_barrier()
    pltpu.sync_copy(dst_hbm.at[w], dst_v)

    def body(g, carry):
        descs = [
            pltpu.async_copy(ones_v, acc_sh.at[dst_v.at[g * 8 + b]], sem,
                             add=True)
            for b in range(8)
        ]
        for d in descs:
            d.wait()
        return carry

    lax.fori_loop(0, K // 8, body, 0)
    plsc.subcore_barrier()
    pltpu.sync_copy(acc_sh.at[pl.ds(s * CHUNK, CHUNK)],
                    stage_v.at[pl.ds(0, CHUNK)])
    pltpu.sync_copy(stage_v.at[pl.ds(0, CHUNK)],
                    out_hbm.at[pl.ds(c * N_PAD + s * CHUNK, CHUNK)])


@functools.partial(
    pl.kernel,
    out_type=jax.ShapeDtypeStruct((NC, N_PAD, HID), jnp.float32),
    mesh=_MESH,
    scratch_types=[
        pltpu.VMEM((K // 2, LANES), jnp.int32),
        pltpu.VMEM((K // 2, LANES), jnp.int32),
        pltpu.VMEM((LANES, HID), jnp.float32),
        pltpu.VMEM((LANES, HID), jnp.float32),
        pltpu.VMEM_SHARED((N_PAD, HID), jnp.float32),
        pltpu.SemaphoreType.DMA,
        pltpu.SemaphoreType.DMA,
        pltpu.SemaphoreType.DMA,
        pltpu.SemaphoreType.DMA,
    ],
)
def _sc_scatter(tbl_hbm, src_hbm, dst_hbm, zero_hbm, out_hbm,
                src_v, dst_v, rows_a, rows_b, acc_sh, sga, sgb, ssa, ssb):
    c = lax.axis_index("c")
    s = lax.axis_index("s")
    w = s * NC + c
    pltpu.sync_copy(zero_hbm.at[pl.ds(s * CHUNK, CHUNK)],
                    acc_sh.at[pl.ds(s * CHUNK, CHUNK)])
    plsc.subcore_barrier()

    GRP = 10
    KP = K // 2
    bufs = (rows_a, rows_b)
    gsem = (sga, sgb)
    ssem = (ssa, ssb)

    def body(grp, carry):
        j0 = grp * GRP
        g = [None] * GRP
        sd = [None] * GRP
        g[0] = pltpu.async_copy(tbl_hbm.at[src_v.at[j0]], bufs[0], gsem[0])
        g[1] = pltpu.async_copy(tbl_hbm.at[src_v.at[j0 + 1]], bufs[1], gsem[1])
        for b in range(GRP):
            p = b % 2
            g[b].wait()
            sd[b] = pltpu.async_copy(bufs[p], acc_sh.at[dst_v.at[j0 + b]],
                                     ssem[p], add=True)
            if b + 2 < GRP:
                sd[b].wait()
                g[b + 2] = pltpu.async_copy(tbl_hbm.at[src_v.at[j0 + b + 2]],
                                            bufs[p], gsem[p])
        sd[GRP - 2].wait()
        sd[GRP - 1].wait()
        return carry

    for ph in range(2):
        pltpu.sync_copy(src_hbm.at[w, pl.ds(ph * KP, KP)], src_v)
        pltpu.sync_copy(dst_hbm.at[w, pl.ds(ph * KP, KP)], dst_v)
        lax.fori_loop(0, KP // GRP, body, 0)
    plsc.subcore_barrier()
    pltpu.sync_copy(acc_sh.at[pl.ds(s * CHUNK, CHUNK)],
                    out_hbm.at[c, pl.ds(s * CHUNK, CHUNK)])


def _tc_mm2_body(x_ref, wi_ref, w0_ref, hw_ref):
    h0 = jnp.dot(x_ref[...], wi_ref[...], preferred_element_type=jnp.float32)
    hw_ref[...] = jnp.dot(h0, w0_ref[...], preferred_element_type=jnp.float32)


def _tc_scale_body(deg_ref, hw_ref, dis_ref, hws_ref):
    p = deg_ref[...]
    deg_col = p[0, :, :1] + p[1, :, :1]
    dis = lax.rsqrt(1.0 + deg_col)
    dis_ref[...] = dis
    hws_ref[...] = hw_ref[...] * dis[:N, :]


def _tc_mid_body(acc_ref, self_ref, dis_ref, b_ref, w_ref, out_ref):
    a = acc_ref[0, :N, :] + acc_ref[1, :N, :] + self_ref[...]
    dis = dis_ref[:N, :]
    h = jnp.maximum(a * dis + b_ref[...], 0.0)
    hw = jnp.dot(h, w_ref[...], preferred_element_type=jnp.float32)
    out_ref[...] = hw * dis


def _tc_out_body(acc_ref, self_ref, dis_ref, b_ref, w_ref, out_ref):
    a = acc_ref[0, :N, :] + acc_ref[1, :N, :] + self_ref[...]
    h = jnp.maximum(a * dis_ref[:N, :] + b_ref[...], 0.0)
    z = jnp.dot(h, w_ref[...], preferred_element_type=jnp.float32)
    m = jnp.max(z, axis=1, keepdims=True)
    lse = jnp.log(jnp.sum(jnp.exp(z - m), axis=1, keepdims=True))
    out_ref[...] = z - m - lse


def kernel(x, edge_index, W_in, W0, b0, W1, b1, W_out):
    src = edge_index[0]
    dst = edge_index[1]
    pad = E_PAD - E
    pad_ar = jnp.arange(pad, dtype=jnp.int32)
    pad_src = (pad_ar * 127) % N
    pad_dst = N + pad_ar % (N_PAD - N)
    src_b = jnp.concatenate([src, pad_src]).reshape(NW, K, LANES)
    dst_b = jnp.concatenate([dst, pad_dst]).reshape(NW, K, LANES)
    zeros2 = jnp.zeros((N_PAD, HID), jnp.float32)

    hw0_raw = pl.pallas_call(
        _tc_mm2_body,
        out_shape=jax.ShapeDtypeStruct((N, HID), jnp.float32),
    )(x, W_in, W0)

    deg_p = _sc_degree(dst_b).reshape(NC, N_PAD, 1)

    dis, hw0 = pl.pallas_call(
        _tc_scale_body,
        out_shape=(jax.ShapeDtypeStruct((N_PAD, 1), jnp.float32),
                   jax.ShapeDtypeStruct((N, HID), jnp.float32)),
    )(deg_p, hw0_raw)

    acc1 = _sc_scatter(hw0, src_b, dst_b, zeros2)

    hw1 = pl.pallas_call(
        _tc_mid_body,
        out_shape=jax.ShapeDtypeStruct((N, HID), jnp.float32),
    )(acc1, hw0, dis, b0.reshape(1, HID), W1)

    acc2 = _sc_scatter(hw1, src_b, dst_b, zeros2)

    out = pl.pallas_call(
        _tc_out_body,
        out_shape=jax.ShapeDtypeStruct((N, C), jnp.float32),
    )(acc2, hw1, dis, b1.reshape(1, HID), W_out)

    return out
